# Initial kernel scaffold; baseline (speedup 1.0000x reference)
import jax
import jax.numpy as jnp
from jax.experimental import pallas as pl


def kernel(position_embed, positions, position_bias):
    raise NotImplementedError("write your pallas kernel here")


# speedup vs baseline: 1.6095x; 1.6095x over previous
import functools

import jax
import jax.numpy as jnp
from jax import lax
from jax.experimental import pallas as pl
from jax.experimental.pallas import tpu as pltpu
from jax.experimental.pallas import tpu_sc as plsc

B = 16384
E = 128
NC = 2
NS = 16
NW = NC * NS
ROWS_PER_W = B // NW
CHUNK = 128
NCHUNK = ROWS_PER_W // CHUNK
LANES = 16

_mesh = plsc.VectorSubcoreMesh(core_axis_name="c", subcore_axis_name="s")


@functools.partial(
    pl.kernel,
    mesh=_mesh,
    out_type=jax.ShapeDtypeStruct((B, E), jnp.float32),
    scratch_types=[
        pltpu.VMEM((CHUNK,), jnp.int32),
        pltpu.VMEM((CHUNK, E), jnp.float32),
        pltpu.VMEM((CHUNK, E), jnp.float32),
        pltpu.SemaphoreType.DMA,
    ],
)
def _sc_bias_add(embed_hbm, pos_hbm, table_hbm, out_hbm, idx_v, bias_v, acc_v, sem):
    wid = lax.axis_index("s") * NC + lax.axis_index("c")
    base = wid * ROWS_PER_W
    for c in range(NCHUNK):
        off = base + c * CHUNK
        pltpu.sync_copy(pos_hbm.at[pl.ds(off, CHUNK)], idx_v)
        gather = pltpu.async_copy(table_hbm.at[idx_v], bias_v, sem)
        pltpu.sync_copy(embed_hbm.at[pl.ds(off, CHUNK)], acc_v)
        gather.wait()

        def add_row(r, _):
            for j in range(E // LANES):
                sl = pl.ds(j * LANES, LANES)
                acc_v[r, sl] = acc_v[r, sl] + bias_v[r, sl]
            return 0

        lax.fori_loop(0, CHUNK, add_row, 0)
        pltpu.sync_copy(acc_v, out_hbm.at[pl.ds(off, CHUNK)])


def kernel(position_embed, positions, position_bias):
    return _sc_bias_add(position_embed, positions.astype(jnp.int32), position_bias)


# speedup vs baseline: 1.6159x; 1.0040x over previous
import functools

import jax
import jax.numpy as jnp
from jax import lax
from jax.experimental import pallas as pl
from jax.experimental.pallas import tpu as pltpu
from jax.experimental.pallas import tpu_sc as plsc

B = 16384
E = 128
NC = 2
NS = 16
NW = NC * NS
ROWS_PER_W = B // NW
CHUNK = 128
NCHUNK = ROWS_PER_W // CHUNK
LANES = 16

_mesh = plsc.VectorSubcoreMesh(core_axis_name="c", subcore_axis_name="s")


@functools.partial(
    pl.kernel,
    mesh=_mesh,
    out_type=jax.ShapeDtypeStruct((B, E), jnp.float32),
    scratch_types=[
        pltpu.VMEM((CHUNK,), jnp.int32),
        pltpu.VMEM((CHUNK, E), jnp.float32),
        pltpu.VMEM((CHUNK, E), jnp.float32),
        pltpu.SemaphoreType.DMA,
    ],
)
def _sc_bias_add(embed_hbm, pos_hbm, table_hbm, out_hbm, idx_v, bias_v, acc_v, sem):
    wid = lax.axis_index("s") * NC + lax.axis_index("c")
    base = wid * ROWS_PER_W
    for c in range(NCHUNK):
        off = base + c * CHUNK
        pltpu.sync_copy(pos_hbm.at[pl.ds(off, CHUNK)], idx_v)
        pltpu.sync_copy(embed_hbm.at[pl.ds(off, CHUNK)], acc_v)
        pltpu.async_copy(table_hbm.at[idx_v], acc_v, sem, add=True).wait()
        pltpu.sync_copy(acc_v, out_hbm.at[pl.ds(off, CHUNK)])


def kernel(position_embed, positions, position_bias):
    return _sc_bias_add(position_embed, positions.astype(jnp.int32), position_bias)
